# Initial kernel scaffold; baseline (speedup 1.0000x reference)
import jax
import jax.numpy as jnp
from jax.experimental import pallas as pl


def kernel(root_indices, compat_matrix):
    raise NotImplementedError("write your pallas kernel here")


# speedup vs baseline: 1.0113x; 1.0113x over previous
import functools

import jax
import jax.numpy as jnp
from jax import lax
from jax.experimental import pallas as pl
from jax.experimental.pallas import tpu as pltpu
from jax.experimental.pallas import tpu_sc as plsc

B = 4096
N = 8192
NC = 2
NS = 16
L = 16
NW = NC * NS
RPW = B // NW
K = 4
NCHUNK = RPW // K
JBLK = B // L

_mesh = plsc.VectorSubcoreMesh(core_axis_name="c", subcore_axis_name="s")


@functools.partial(
    pl.kernel,
    out_type=jax.ShapeDtypeStruct((B, B), jnp.float32),
    mesh=_mesh,
    compiler_params=pltpu.CompilerParams(needs_layout_passes=False,
                                         use_tc_tiling_on_sc=False),
    scratch_types=[
        pltpu.VMEM((B,), jnp.int32),
        pltpu.VMEM((NCHUNK * 8,), jnp.int32),
        pltpu.VMEM((2 * K, N), jnp.float32),
        pltpu.VMEM((2 * K, B), jnp.float32),
        pltpu.SemaphoreType.DMA,
        pltpu.SemaphoreType.DMA,
        pltpu.SemaphoreType.DMA,
        pltpu.SemaphoreType.DMA,
    ],
)
def _pairwise_sc(ri_hbm, ri_chunks_hbm, compat_hbm, out_hbm,
                 ri_v, idxs_v, rowbuf, outbuf, gsem0, gsem1, osem0, osem1):
    wid = lax.axis_index("s") * NC + lax.axis_index("c")
    row0 = wid * RPW
    gsems = (gsem0, gsem1)
    osems = (osem0, osem1)

    pltpu.sync_copy(ri_hbm, ri_v)
    pltpu.sync_copy(ri_chunks_hbm.at[pl.ds(wid * NCHUNK * 8, NCHUNK * 8)],
                    idxs_v)

    def start_gather(c, q):
        idx_ref = idxs_v.at[pl.ds(c * 8, K)]
        return pltpu.async_copy(compat_hbm.at[idx_ref],
                                rowbuf.at[pl.ds(q * K, K)], gsems[q])

    def compute_chunk(c, p):
        def body(jb, carry):
            cidx = ri_v[pl.ds(jb * L, L)]
            for r in range(K):
                ridx = jnp.full((L,), p * K + r, dtype=jnp.int32)
                vals = plsc.load_gather(rowbuf, [ridx, cidx])
                outbuf[p * K + r, pl.ds(jb * L, L)] = vals
            return carry
        lax.fori_loop(0, JBLK, body, 0, unroll=2)

    ghandles = [None, None]
    ohandles = [None, None]
    ghandles[0] = start_gather(0, 0)
    for c in range(NCHUNK):
        p = c % 2
        if c + 1 < NCHUNK:
            ghandles[1 - p] = start_gather(c + 1, 1 - p)
        ghandles[p].wait()
        if ohandles[p] is not None:
            ohandles[p].wait()
        compute_chunk(c, p)
        ohandles[p] = pltpu.async_copy(outbuf.at[pl.ds(p * K, K)],
                                       out_hbm.at[pl.ds(row0 + c * K, K)],
                                       osems[p])
    ohandles[0].wait()
    ohandles[1].wait()


def kernel(root_indices, compat_matrix):
    ri = root_indices.astype(jnp.int32)
    chunks = ri.reshape(-1, K)
    ri_chunks = jnp.concatenate([chunks, jnp.zeros_like(chunks)],
                                axis=1).reshape(-1)
    return _pairwise_sc(ri, ri_chunks, compat_matrix)


# speedup vs baseline: 1.4736x; 1.4571x over previous
import functools

import jax
import jax.numpy as jnp
from jax import lax
from jax.experimental import pallas as pl
from jax.experimental.pallas import tpu as pltpu
from jax.experimental.pallas import tpu_sc as plsc

B = 4096
N = 8192
NC = 2
NS = 16
L = 16
NW = NC * NS
RPW = B // NW
K = 2
NCHUNK = RPW // K
LPR = N // 128
GROUP = 8 // K
NGROUP = NCHUNK // GROUP

_mesh = plsc.VectorSubcoreMesh(core_axis_name="c", subcore_axis_name="s")


@functools.partial(
    pl.kernel,
    out_type=jax.ShapeDtypeStruct((B // 8, B // 128, 8, 128), jnp.float32),
    mesh=_mesh,
    compiler_params=pltpu.CompilerParams(needs_layout_passes=False,
                                         use_tc_tiling_on_sc=False),
    scratch_types=[
        pltpu.VMEM((B,), jnp.int32),
        pltpu.VMEM((B,), jnp.int32),
        pltpu.VMEM((NCHUNK, K * LPR), jnp.int32),
        pltpu.VMEM((2 * K * LPR, 128), jnp.float32),
        pltpu.VMEM((2, B // 128, 8, 128), jnp.float32),
        pltpu.SemaphoreType.DMA,
        pltpu.SemaphoreType.DMA,
        pltpu.SemaphoreType.DMA,
        pltpu.SemaphoreType.DMA,
    ],
)
def _pairwise_sc(chi_hbm, clo_hbm, gidx_hbm, compat_hbm, out_hbm,
                 chi_v, clo_v, gidx_v, rowbuf, outbuf,
                 gsem0, gsem1, osem0, osem1):
    wid = lax.axis_index("s") * NC + lax.axis_index("c")
    gsems = (gsem0, gsem1)
    osems = (osem0, osem1)

    pltpu.sync_copy(chi_hbm, chi_v)
    pltpu.sync_copy(clo_hbm, clo_v)
    pltpu.sync_copy(gidx_hbm.at[pl.ds(wid * NCHUNK, NCHUNK)], gidx_v)

    def start_gather(c, q):
        return pltpu.async_copy(compat_hbm.at[gidx_v.at[c]],
                                rowbuf.at[pl.ds(q * K * LPR, K * LPR)],
                                gsems[q])

    def compute_chunk(c, q, p, slot):
        def body(t, carry):
            for u in range(8):
                hi = chi_v[pl.ds(t * 128 + u * L, L)]
                lo = clo_v[pl.ds(t * 128 + u * L, L)]
                for r in range(K):
                    idx0 = hi + (q * K * LPR + r * LPR)
                    vals = plsc.load_gather(rowbuf, [idx0, lo])
                    outbuf[p, t, slot * K + r, pl.ds(u * L, L)] = vals
            return carry
        lax.fori_loop(0, B // 128, body, 0)

    ghandles = [None, None]
    ohandles = [None, None]
    ghandles[0] = start_gather(0, 0)
    for c in range(NCHUNK):
        q = c % 2
        g = c // GROUP
        p = g % 2
        slot = c % GROUP
        if c + 1 < NCHUNK:
            ghandles[1 - q] = start_gather(c + 1, 1 - q)
        ghandles[q].wait()
        if slot == 0 and ohandles[p] is not None:
            ohandles[p].wait()
        compute_chunk(c, q, p, slot)
        if slot == GROUP - 1:
            ohandles[p] = pltpu.async_copy(outbuf.at[p],
                                           out_hbm.at[wid * NGROUP + g],
                                           osems[p])
    ohandles[0].wait()
    ohandles[1].wait()


def kernel(root_indices, compat_matrix):
    ri = root_indices.astype(jnp.int32)
    compat_lines = (compat_matrix.reshape(N // 8, 8, N // 128, 128)
                    .transpose(0, 2, 1, 3).reshape(N * N // 128, 128))
    chi = ri >> 7
    clo = ri & 127
    t8 = jnp.arange(LPR, dtype=jnp.int32) * 8
    gidx = ((ri >> 3) * 512 + (ri & 7))[:, None] + t8[None, :]
    gidx = gidx.reshape(B // K, K * LPR)
    out4 = _pairwise_sc(chi, clo, gidx, compat_lines)
    return out4.transpose(0, 2, 1, 3).reshape(B, B)


# speedup vs baseline: 4.3556x; 2.9557x over previous
import functools

import jax
import jax.numpy as jnp
from jax import lax
from jax.experimental import pallas as pl
from jax.experimental.pallas import tpu as pltpu
from jax.experimental.pallas import tpu_sc as plsc

B = 4096
N = 8192
NC = 2
NS = 16
L = 16
NW = NC * NS
RPW = B // NW
K = 4
NCHUNK = RPW // K
LPR = N // 128
NBLK = RPW // 8

_mesh = plsc.VectorSubcoreMesh(core_axis_name="c", subcore_axis_name="s")


@functools.partial(
    pl.kernel,
    out_type=jax.ShapeDtypeStruct((B // 8, B // 128, 8, 128), jnp.float32),
    mesh=_mesh,
    compiler_params=pltpu.CompilerParams(needs_layout_passes=False,
                                         use_tc_tiling_on_sc=False),
    scratch_types=[
        pltpu.VMEM((B,), jnp.int32),
        pltpu.VMEM((B,), jnp.int32),
        pltpu.VMEM((NCHUNK, K * LPR), jnp.int32),
        pltpu.VMEM((2 * K * LPR, 128), jnp.float32),
        pltpu.VMEM((2, B // 128, K, 128), jnp.float32),
        pltpu.SemaphoreType.DMA,
        pltpu.SemaphoreType.DMA,
        pltpu.SemaphoreType.DMA,
        pltpu.SemaphoreType.DMA,
    ],
)
def _pairwise_sc(chi_hbm, clo_hbm, gidx_hbm, compat_hbm, out_hbm,
                 chi_v, clo_v, gidx_v, rowbuf, outbuf,
                 gsem0, gsem1, osem0, osem1):
    wid = lax.axis_index("s") * NC + lax.axis_index("c")
    gsems = (gsem0, gsem1)
    osems = (osem0, osem1)

    pltpu.sync_copy(chi_hbm, chi_v)
    pltpu.sync_copy(clo_hbm, clo_v)
    pltpu.sync_copy(gidx_hbm.at[pl.ds(wid * NCHUNK, NCHUNK)], gidx_v)

    def start_gather(c, q):
        return pltpu.async_copy(compat_hbm.at[gidx_v.at[c]],
                                rowbuf.at[pl.ds(q * K * LPR, K * LPR)],
                                gsems[q])

    def compute_chunk(q, p):
        @plsc.parallel_loop(0, B // 128, unroll=2)
        def body(t):
            for u in range(8):
                hi = chi_v[pl.ds(t * 128 + u * L, L)]
                lo = clo_v[pl.ds(t * 128 + u * L, L)]
                for r in range(K):
                    idx0 = hi + (q * K * LPR + r * LPR)
                    vals = plsc.load_gather(rowbuf, [idx0, lo])
                    outbuf[p, t, r, pl.ds(u * L, L)] = vals

    def out_dst(blk, q):
        return out_hbm.at[wid * NBLK + blk, :, pl.ds(q * K, K)]

    start_gather(0, 0)

    def outer(cc, carry):
        for q in range(2):
            c = 2 * cc + q

            @pl.when(c + 1 < NCHUNK)
            def _():
                start_gather(c + 1, 1 - q)

            pltpu.make_async_copy(compat_hbm.at[gidx_v.at[c]],
                                  rowbuf.at[pl.ds(q * K * LPR, K * LPR)],
                                  gsems[q]).wait()

            @pl.when(c >= 2)
            def _():
                pltpu.make_async_copy(outbuf.at[q], out_dst(cc - 1, q),
                                      osems[q]).wait()

            compute_chunk(q, q)
            pltpu.async_copy(outbuf.at[q], out_dst(cc, q), osems[q])
        return carry

    lax.fori_loop(0, NCHUNK // 2, outer, 0)
    for q in range(2):
        pltpu.make_async_copy(outbuf.at[q], out_dst(NCHUNK // 2 - 1, q),
                              osems[q]).wait()


def kernel(root_indices, compat_matrix):
    ri = root_indices.astype(jnp.int32)
    compat_lines = (compat_matrix.reshape(N // 8, 8, N // 128, 128)
                    .transpose(0, 2, 1, 3).reshape(N * N // 128, 128))
    chi = ri >> 7
    clo = ri & 127
    t8 = jnp.arange(LPR, dtype=jnp.int32) * 8
    gidx = ((ri >> 3) * 512 + (ri & 7))[:, None] + t8[None, :]
    gidx = gidx.reshape(B // K, K * LPR)
    out4 = _pairwise_sc(chi, clo, gidx, compat_lines)
    return out4.transpose(0, 2, 1, 3).reshape(B, B)
